# baseline (speedup 1.0000x reference)
import functools

import jax
import jax.numpy as jnp
from jax import lax
from jax.experimental import pallas as pl
from jax.experimental.pallas import tpu as pltpu
from jax.experimental.pallas import tpu_sc as plsc

B_SZ = 1024
L = 50
H = 64
D_STATE = 32
D_CONV = 4
D_INNER = 128
N_EXP = 4


_NC = 2
_NS = 16
_NW = _NC * _NS
_R = 3 * B_SZ * L
_RPW = _R // _NW
_CHUNK = 960


def _gather_sc(table, idx):
    mesh = plsc.VectorSubcoreMesh(core_axis_name="c", subcore_axis_name="s")

    @functools.partial(
        pl.kernel,
        mesh=mesh,
        out_type=jax.ShapeDtypeStruct((_R, H), jnp.float32),
        scratch_types=[
            pltpu.VMEM((_CHUNK,), jnp.int32),
            pltpu.VMEM((_CHUNK, H), jnp.float32),
            pltpu.SemaphoreType.DMA,
        ],
        compiler_params=pltpu.CompilerParams(use_tc_tiling_on_sc=False),
    )
    def gk(table_hbm, idx_hbm, out_hbm, idx_v, rows_v, sem):
        wid = lax.axis_index("s") * _NC + lax.axis_index("c")
        base = wid * _RPW
        for ci in range(_RPW // _CHUNK):
            off = ci * _CHUNK
            pltpu.sync_copy(idx_hbm.at[pl.ds(base + off, _CHUNK)], idx_v)
            pltpu.async_copy(table_hbm.at[idx_v], rows_v, sem).wait()
            pltpu.sync_copy(rows_v, out_hbm.at[pl.ds(base + off, _CHUNK)])

    return gk(table, idx)


_BB = 128


def _tc_body(seq_ref, pose_ref, nege_ref, pemb_ref, win_ref, cwt_ref, cb_ref,
             wx4_ref, wdt_ref, bdt_ref, wxbc_ref, at_ref, dd_ref, wout_ref, gw_ref,
             gb_ref, wexp_ref, bexp_ref, pos_o_ref, neg_o_ref,
             ys_ref, dt_s, u_s, b_s):
    f32 = jnp.float32
    x = seq_ref[...] * (H ** 0.5) + pemb_ref[...]
    xz = jnp.dot(x.reshape(L * _BB, H), win_ref[...],
                 preferred_element_type=f32).reshape(L, _BB, 2 * D_INNER)
    x_in = xz[:, :, :D_INNER]
    z = xz[:, :, D_INNER:]

    cw = cwt_ref[...]
    acc = jnp.broadcast_to(cb_ref[...].reshape(1, 1, D_INNER), (L, _BB, D_INNER))
    for k in range(D_CONV):
        s = D_CONV - 1 - k
        if s == 0:
            seg = x_in
        else:
            seg = jnp.concatenate(
                [jnp.zeros((s, _BB, D_INNER), f32), x_in[:L - s]], axis=0)
        acc = acc + cw[k].reshape(1, 1, D_INNER) * seg
    xc = acc * jax.nn.sigmoid(acc)

    xcf = xc.reshape(L * _BB, D_INNER)
    dt_lo = jnp.dot(xcf, wx4_ref[...], preferred_element_type=f32)
    dt = jax.nn.softplus(
        jnp.dot(dt_lo, wdt_ref[...], preferred_element_type=f32) + bdt_ref[...])
    bc = jnp.dot(xcf, wxbc_ref[...],
                 preferred_element_type=f32).reshape(L, _BB, 2 * D_STATE)
    b_s[:, :D_STATE] = jnp.transpose(bc[:, :, :D_STATE], (0, 2, 1))
    b_s[:, D_STATE:] = jnp.transpose(bc[:, :, D_STATE:], (0, 2, 1))
    dt_s[...] = dt.reshape(L, _BB, D_INNER)
    u_s[...] = xc
    at = at_ref[...]

    def one_step(t, h):
        dt_t = dt_s[pl.ds(t, 1)][0]
        u_t = u_s[pl.ds(t, 1)][0]
        bc_t = b_s[pl.ds(t, 1)][0]
        b_t = bc_t[:D_STATE]
        c_t = bc_t[D_STATE:]
        dA = jnp.exp(dt_t[None, :, :] * at[:, None, :])
        h = dA * h + (dt_t[None, :, :] * b_t[:, :, None]) * u_t[None, :, :]
        y = jnp.sum(h * c_t[:, :, None], axis=0)
        ys_ref[pl.ds(t, 1)] = y[None]
        return h

    def step25(i, h):
        t = i * 25
        for j in range(25):
            h = one_step(t + j, h)
        return h

    lax.fori_loop(0, L // 25, step25, jnp.zeros((D_STATE, _BB, D_INNER), f32))

    y = ys_ref[...] + xc * dd_ref[...].reshape(1, 1, D_INNER)
    y = y * (z * jax.nn.sigmoid(z))
    feats = jnp.dot(y.reshape(L * _BB, D_INNER), wout_ref[...],
                    preferred_element_type=f32)

    gt = lax.dot_general(gw_ref[...], feats, (((0,), (1,)), ((), ())),
                         preferred_element_type=f32)
    gt = gt + jnp.transpose(gb_ref[...])
    m = jnp.max(gt, axis=0, keepdims=True)
    ex = jnp.exp(gt - m)
    sm = ex / jnp.sum(ex, axis=0, keepdims=True)
    msm = jnp.max(sm, axis=0, keepdims=True)
    ism = sm == msm
    run = ism[0:1]
    wrows = [jnp.where(run, msm, 0.0)]
    for e in range(1, N_EXP):
        fe = ism[e:e + 1] & (~run)
        run = run | fe
        wrows.append(jnp.where(fe, msm, 0.0))
    wsel_t = jnp.concatenate(wrows, axis=0)
    wsel = jnp.transpose(wsel_t)
    col = lax.broadcasted_iota(jnp.int32, (N_EXP, N_EXP * H), 1)
    row = lax.broadcasted_iota(jnp.int32, (N_EXP, N_EXP * H), 0)
    ones_blk = jnp.where(col // H == row, 1.0, 0.0)
    bcast = jnp.dot(wsel, ones_blk, preferred_element_type=f32)
    eo = jnp.dot(feats, wexp_ref[...], preferred_element_type=f32) + bexp_ref[...]
    prod = bcast * eo
    out = (prod[:, 0:H] + prod[:, H:2 * H]) + (prod[:, 2 * H:3 * H]
                                               + prod[:, 3 * H:4 * H])
    moe = out.reshape(L, _BB, H)

    pos_o_ref[...] = jnp.sum(moe * pose_ref[...], axis=-1)[None]
    neg_o_ref[...] = jnp.sum(moe * nege_ref[...], axis=-1)[None]


def _dense_tc(seq_e, pos_e, neg_e, pemb, win, cwt, cb, wx4, wdt, bdt, wxbc,
              at, dd, wout, gw, gb, wexp, bexp):
    grid = (B_SZ // _BB,)
    big = pl.BlockSpec((L, _BB, H), lambda i: (0, i, 0))
    full = lambda shape: pl.BlockSpec(shape, lambda i: tuple(0 for _ in shape))
    outs = pl.BlockSpec((1, L, _BB), lambda i: (i, 0, 0))
    return pl.pallas_call(
        _tc_body,
        grid=grid,
        in_specs=[
            big, big, big,
            full((L, 1, H)),
            full((H, 2 * D_INNER)),
            full((D_CONV, D_INNER)),
            full((1, D_INNER)),
            full((D_INNER, 8)),
            full((8, D_INNER)),
            full((1, D_INNER)),
            full((D_INNER, 2 * D_STATE)),
            full((D_STATE, D_INNER)),
            full((1, D_INNER)),
            full((D_INNER, H)),
            full((H, N_EXP)),
            full((1, N_EXP)),
            full((H, N_EXP * H)),
            full((1, N_EXP * H)),
        ],
        out_specs=[outs, outs],
        out_shape=[jax.ShapeDtypeStruct((B_SZ // _BB, L, _BB), jnp.float32)] * 2,
        scratch_shapes=[pltpu.VMEM((L, _BB, D_INNER), jnp.float32),
                        pltpu.VMEM((L, _BB, D_INNER), jnp.float32),
                        pltpu.VMEM((L, _BB, D_INNER), jnp.float32),
                        pltpu.VMEM((L, 2 * D_STATE, _BB), jnp.float32)],
        compiler_params=pltpu.CompilerParams(
            dimension_semantics=("arbitrary",)),
    )(seq_e, pos_e, neg_e, pemb, win, cwt, cb, wx4, wdt, bdt, wxbc, at, dd,
      wout, gw, gb, wexp, bexp)


def kernel(user_ids, log_seqs, pos_seqs, neg_seqs, params):
    p = params
    idx = jnp.concatenate([
        log_seqs.T.reshape(-1), pos_seqs.T.reshape(-1), neg_seqs.T.reshape(-1)
    ]).astype(jnp.int32)
    rows = _gather_sc(p['item_emb'], idx)
    n = B_SZ * L
    seq_e = rows[:n].reshape(L, B_SZ, H)
    pos_e = rows[n:2 * n].reshape(L, B_SZ, H)
    neg_e = rows[2 * n:].reshape(L, B_SZ, H)

    dt_rank = p['W_dt'].shape[0]
    wx4 = jnp.pad(p['W_x'][:, :dt_rank], ((0, 0), (0, 8 - dt_rank)))
    wdt = jnp.pad(p['W_dt'], ((0, 8 - dt_rank), (0, 0)))
    wxbc = p['W_x'][:, dt_rank:]
    at = (-jnp.exp(p['A_log'])).T
    wexp = jnp.transpose(p['exp_W'], (1, 0, 2)).reshape(H, N_EXP * H)
    bexp = p['exp_b'].reshape(1, N_EXP * H)

    pos_l, neg_l = _dense_tc(
        seq_e, pos_e, neg_e,
        p['pos_emb'][:L].reshape(L, 1, H),
        p['W_in'],
        p['conv_w'].T,
        p['conv_b'].reshape(1, D_INNER),
        wx4,
        wdt,
        p['b_dt'].reshape(1, D_INNER),
        wxbc,
        at,
        p['D'].reshape(1, D_INNER),
        p['W_out'],
        p['gate_W'],
        p['gate_b'].reshape(1, N_EXP),
        wexp,
        bexp,
    )
    return (jnp.transpose(pos_l, (0, 2, 1)).reshape(B_SZ, L),
            jnp.transpose(neg_l, (0, 2, 1)).reshape(B_SZ, L))


# speedup vs baseline: 1.0921x; 1.0921x over previous
import functools

import jax
import jax.numpy as jnp
from jax import lax
from jax.experimental import pallas as pl
from jax.experimental.pallas import tpu as pltpu
from jax.experimental.pallas import tpu_sc as plsc

B_SZ = 1024
L = 50
H = 64
D_STATE = 32
D_CONV = 4
D_INNER = 128
N_EXP = 4


_NC = 2
_NS = 16
_NW = _NC * _NS
_R = 3 * B_SZ * L
_RPW = _R // _NW
_CHUNK = 960


def _gather_sc(table, idx):
    mesh = plsc.VectorSubcoreMesh(core_axis_name="c", subcore_axis_name="s")

    @functools.partial(
        pl.kernel,
        mesh=mesh,
        out_type=jax.ShapeDtypeStruct((_R, H), jnp.float32),
        scratch_types=[
            pltpu.VMEM((_CHUNK,), jnp.int32),
            pltpu.VMEM((_CHUNK, H), jnp.float32),
            pltpu.SemaphoreType.DMA,
        ],
        compiler_params=pltpu.CompilerParams(use_tc_tiling_on_sc=False),
    )
    def gk(table_hbm, idx_hbm, out_hbm, idx_v, rows_v, sem):
        wid = lax.axis_index("s") * _NC + lax.axis_index("c")
        base = wid * _RPW
        for ci in range(_RPW // _CHUNK):
            off = ci * _CHUNK
            pltpu.sync_copy(idx_hbm.at[pl.ds(base + off, _CHUNK)], idx_v)
            pltpu.async_copy(table_hbm.at[idx_v], rows_v, sem).wait()
            pltpu.sync_copy(rows_v, out_hbm.at[pl.ds(base + off, _CHUNK)])

    return gk(table, idx)


_BB = 64


def _tc_body(seq_ref, pose_ref, nege_ref, pemb_ref, win_ref, cwt_ref, cb_ref,
             wx4_ref, wdt_ref, bdt_ref, wxbc_ref, at_ref, dd_ref, wout_ref, gw_ref,
             gb_ref, wexp_ref, bexp_ref, pos_o_ref, neg_o_ref,
             ys_ref, dt_s, u_s, b_s):
    f32 = jnp.float32
    x = seq_ref[...] * (H ** 0.5) + pemb_ref[...]
    xz = jnp.dot(x.reshape(L * _BB, H), win_ref[...],
                 preferred_element_type=f32).reshape(L, _BB, 2 * D_INNER)
    x_in = xz[:, :, :D_INNER]
    z = xz[:, :, D_INNER:]

    cw = cwt_ref[...]
    acc = jnp.broadcast_to(cb_ref[...].reshape(1, 1, D_INNER), (L, _BB, D_INNER))
    for k in range(D_CONV):
        s = D_CONV - 1 - k
        if s == 0:
            seg = x_in
        else:
            seg = jnp.concatenate(
                [jnp.zeros((s, _BB, D_INNER), f32), x_in[:L - s]], axis=0)
        acc = acc + cw[k].reshape(1, 1, D_INNER) * seg
    xc = acc * jax.nn.sigmoid(acc)

    xcf = xc.reshape(L * _BB, D_INNER)
    dt_lo = jnp.dot(xcf, wx4_ref[...], preferred_element_type=f32)
    dt = jax.nn.softplus(
        jnp.dot(dt_lo, wdt_ref[...], preferred_element_type=f32) + bdt_ref[...])
    bc = jnp.dot(xcf, wxbc_ref[...],
                 preferred_element_type=f32).reshape(L, _BB, 2 * D_STATE)
    b_s[:, :D_STATE] = jnp.transpose(bc[:, :, :D_STATE], (0, 2, 1))
    b_s[:, D_STATE:] = jnp.transpose(bc[:, :, D_STATE:], (0, 2, 1))
    dt_s[...] = dt.reshape(L, _BB, D_INNER)
    u_s[...] = xc
    at = at_ref[...]

    def one_step(t, h):
        dt_t = dt_s[pl.ds(t, 1)][0]
        u_t = u_s[pl.ds(t, 1)][0]
        bc_t = b_s[pl.ds(t, 1)][0]
        b_t = bc_t[:D_STATE]
        c_t = bc_t[D_STATE:]
        dA = jnp.exp(dt_t[None, :, :] * at[:, None, :])
        h = dA * h + (dt_t[None, :, :] * b_t[:, :, None]) * u_t[None, :, :]
        y = jnp.sum(h * c_t[:, :, None], axis=0)
        ys_ref[pl.ds(t, 1)] = y[None]
        return h

    def step10(i, h):
        t = i * 10
        for j in range(10):
            h = one_step(t + j, h)
        return h

    lax.fori_loop(0, L // 10, step10, jnp.zeros((D_STATE, _BB, D_INNER), f32))

    y = ys_ref[...] + xc * dd_ref[...].reshape(1, 1, D_INNER)
    y = y * (z * jax.nn.sigmoid(z))
    feats = jnp.dot(y.reshape(L * _BB, D_INNER), wout_ref[...],
                    preferred_element_type=f32)

    gt = lax.dot_general(gw_ref[...], feats, (((0,), (1,)), ((), ())),
                         preferred_element_type=f32)
    gt = gt + jnp.transpose(gb_ref[...])
    m = jnp.max(gt, axis=0, keepdims=True)
    ex = jnp.exp(gt - m)
    sm = ex / jnp.sum(ex, axis=0, keepdims=True)
    msm = jnp.max(sm, axis=0, keepdims=True)
    ism = sm == msm
    run = ism[0:1]
    wrows = [jnp.where(run, msm, 0.0)]
    for e in range(1, N_EXP):
        fe = ism[e:e + 1] & (~run)
        run = run | fe
        wrows.append(jnp.where(fe, msm, 0.0))
    wsel_t = jnp.concatenate(wrows, axis=0)
    wsel = jnp.transpose(wsel_t)
    col = lax.broadcasted_iota(jnp.int32, (N_EXP, N_EXP * H), 1)
    row = lax.broadcasted_iota(jnp.int32, (N_EXP, N_EXP * H), 0)
    ones_blk = jnp.where(col // H == row, 1.0, 0.0)
    bcast = jnp.dot(wsel, ones_blk, preferred_element_type=f32)
    eo = jnp.dot(feats, wexp_ref[...], preferred_element_type=f32) + bexp_ref[...]
    prod = bcast * eo
    out = (prod[:, 0:H] + prod[:, H:2 * H]) + (prod[:, 2 * H:3 * H]
                                               + prod[:, 3 * H:4 * H])
    moe = out.reshape(L, _BB, H)

    pos_o_ref[...] = jnp.sum(moe * pose_ref[...], axis=-1)[None]
    neg_o_ref[...] = jnp.sum(moe * nege_ref[...], axis=-1)[None]


def _dense_tc(seq_e, pos_e, neg_e, pemb, win, cwt, cb, wx4, wdt, bdt, wxbc,
              at, dd, wout, gw, gb, wexp, bexp):
    grid = (B_SZ // _BB,)
    big = pl.BlockSpec((L, _BB, H), lambda i: (0, i, 0))
    full = lambda shape: pl.BlockSpec(shape, lambda i: tuple(0 for _ in shape))
    outs = pl.BlockSpec((1, L, _BB), lambda i: (i, 0, 0))
    return pl.pallas_call(
        _tc_body,
        grid=grid,
        in_specs=[
            big, big, big,
            full((L, 1, H)),
            full((H, 2 * D_INNER)),
            full((D_CONV, D_INNER)),
            full((1, D_INNER)),
            full((D_INNER, 8)),
            full((8, D_INNER)),
            full((1, D_INNER)),
            full((D_INNER, 2 * D_STATE)),
            full((D_STATE, D_INNER)),
            full((1, D_INNER)),
            full((D_INNER, H)),
            full((H, N_EXP)),
            full((1, N_EXP)),
            full((H, N_EXP * H)),
            full((1, N_EXP * H)),
        ],
        out_specs=[outs, outs],
        out_shape=[jax.ShapeDtypeStruct((B_SZ // _BB, L, _BB), jnp.float32)] * 2,
        scratch_shapes=[pltpu.VMEM((L, _BB, D_INNER), jnp.float32),
                        pltpu.VMEM((L, _BB, D_INNER), jnp.float32),
                        pltpu.VMEM((L, _BB, D_INNER), jnp.float32),
                        pltpu.VMEM((L, 2 * D_STATE, _BB), jnp.float32)],
        compiler_params=pltpu.CompilerParams(
            dimension_semantics=("arbitrary",)),
    )(seq_e, pos_e, neg_e, pemb, win, cwt, cb, wx4, wdt, bdt, wxbc, at, dd,
      wout, gw, gb, wexp, bexp)


def kernel(user_ids, log_seqs, pos_seqs, neg_seqs, params):
    p = params
    idx = jnp.concatenate([
        log_seqs.T.reshape(-1), pos_seqs.T.reshape(-1), neg_seqs.T.reshape(-1)
    ]).astype(jnp.int32)
    rows = _gather_sc(p['item_emb'], idx)
    n = B_SZ * L
    seq_e = rows[:n].reshape(L, B_SZ, H)
    pos_e = rows[n:2 * n].reshape(L, B_SZ, H)
    neg_e = rows[2 * n:].reshape(L, B_SZ, H)

    dt_rank = p['W_dt'].shape[0]
    wx4 = jnp.pad(p['W_x'][:, :dt_rank], ((0, 0), (0, 8 - dt_rank)))
    wdt = jnp.pad(p['W_dt'], ((0, 8 - dt_rank), (0, 0)))
    wxbc = p['W_x'][:, dt_rank:]
    at = (-jnp.exp(p['A_log'])).T
    wexp = jnp.transpose(p['exp_W'], (1, 0, 2)).reshape(H, N_EXP * H)
    bexp = p['exp_b'].reshape(1, N_EXP * H)

    pos_l, neg_l = _dense_tc(
        seq_e, pos_e, neg_e,
        p['pos_emb'][:L].reshape(L, 1, H),
        p['W_in'],
        p['conv_w'].T,
        p['conv_b'].reshape(1, D_INNER),
        wx4,
        wdt,
        p['b_dt'].reshape(1, D_INNER),
        wxbc,
        at,
        p['D'].reshape(1, D_INNER),
        p['W_out'],
        p['gate_W'],
        p['gate_b'].reshape(1, N_EXP),
        wexp,
        bexp,
    )
    return (jnp.transpose(pos_l, (0, 2, 1)).reshape(B_SZ, L),
            jnp.transpose(neg_l, (0, 2, 1)).reshape(B_SZ, L))


# speedup vs baseline: 1.2096x; 1.1076x over previous
import functools

import jax
import jax.numpy as jnp
from jax import lax
from jax.experimental import pallas as pl
from jax.experimental.pallas import tpu as pltpu
from jax.experimental.pallas import tpu_sc as plsc

B_SZ = 1024
L = 50
H = 64
D_STATE = 32
D_CONV = 4
D_INNER = 128
N_EXP = 4


_NC = 2
_NS = 16
_NW = _NC * _NS
_R = 3 * B_SZ * L
_RPW = _R // _NW
_CHUNK = 960


def _gather_sc(table, idx):
    mesh = plsc.VectorSubcoreMesh(core_axis_name="c", subcore_axis_name="s")

    @functools.partial(
        pl.kernel,
        mesh=mesh,
        out_type=jax.ShapeDtypeStruct((_R, H), jnp.float32),
        scratch_types=[
            pltpu.VMEM((_CHUNK,), jnp.int32),
            pltpu.VMEM((_CHUNK,), jnp.int32),
            pltpu.VMEM((_CHUNK, H), jnp.float32),
            pltpu.VMEM((_CHUNK, H), jnp.float32),
            pltpu.SemaphoreType.DMA,
            pltpu.SemaphoreType.DMA,
        ],
        compiler_params=pltpu.CompilerParams(use_tc_tiling_on_sc=False),
    )
    def gk(table_hbm, idx_hbm, out_hbm, idx0, idx1, rows0, rows1, sem0, sem1):
        wid = lax.axis_index("s") * _NC + lax.axis_index("c")
        base = wid * _RPW
        idx_b = (idx0, idx1)
        rows_b = (rows0, rows1)
        sems = (sem0, sem1)
        n = _RPW // _CHUNK
        pltpu.sync_copy(idx_hbm.at[pl.ds(base, _CHUNK)], idx0)
        pend = pltpu.async_copy(table_hbm.at[idx0], rows0, sem0)
        for ci in range(n):
            cur = ci % 2
            nxt = (ci + 1) % 2
            if ci + 1 < n:
                off2 = (ci + 1) * _CHUNK
                pltpu.sync_copy(idx_hbm.at[pl.ds(base + off2, _CHUNK)],
                                idx_b[nxt])
                nxt_pend = pltpu.async_copy(table_hbm.at[idx_b[nxt]],
                                            rows_b[nxt], sems[nxt])
            pend.wait()
            pltpu.sync_copy(rows_b[cur],
                            out_hbm.at[pl.ds(base + ci * _CHUNK, _CHUNK)])
            if ci + 1 < n:
                pend = nxt_pend

    return gk(table, idx)


_BB = 128


def _tc_body(seq_ref, pose_ref, nege_ref, pemb_ref, win_ref, cwt_ref, cb_ref,
             wx4_ref, wdt_ref, bdt_ref, wxbc_ref, at_ref, dd_ref, wout_ref, gw_ref,
             gb_ref, wexp_ref, bexp_ref, pos_o_ref, neg_o_ref,
             ys_ref, dt_s, u_s, b_s):
    f32 = jnp.float32
    x = seq_ref[...] * (H ** 0.5) + pemb_ref[...]
    xz = jnp.dot(x.reshape(L * _BB, H), win_ref[...],
                 preferred_element_type=f32).reshape(L, _BB, 2 * D_INNER)
    x_in = xz[:, :, :D_INNER]
    z = xz[:, :, D_INNER:]

    cw = cwt_ref[...]
    acc = jnp.broadcast_to(cb_ref[...].reshape(1, 1, D_INNER), (L, _BB, D_INNER))
    for k in range(D_CONV):
        s = D_CONV - 1 - k
        if s == 0:
            seg = x_in
        else:
            seg = jnp.concatenate(
                [jnp.zeros((s, _BB, D_INNER), f32), x_in[:L - s]], axis=0)
        acc = acc + cw[k].reshape(1, 1, D_INNER) * seg
    xc = acc * jax.nn.sigmoid(acc)

    xcf = xc.reshape(L * _BB, D_INNER)
    dt_lo = jnp.dot(xcf, wx4_ref[...], preferred_element_type=f32)
    dt = jax.nn.softplus(
        jnp.dot(dt_lo, wdt_ref[...], preferred_element_type=f32) + bdt_ref[...])
    bc = jnp.dot(xcf, wxbc_ref[...],
                 preferred_element_type=f32).reshape(L, _BB, 2 * D_STATE)
    b_s[:, :D_STATE] = jnp.transpose(bc[:, :, :D_STATE], (0, 2, 1))
    b_s[:, D_STATE:] = jnp.transpose(bc[:, :, D_STATE:], (0, 2, 1))
    dt_s[...] = dt.reshape(L, _BB, D_INNER)
    u_s[...] = xc
    at = at_ref[...]

    def one_step(t, h):
        dt_t = dt_s[pl.ds(t, 1)][0]
        u_t = u_s[pl.ds(t, 1)][0]
        bc_t = b_s[pl.ds(t, 1)][0]
        b_t = bc_t[:D_STATE]
        c_t = bc_t[D_STATE:]
        dA = jnp.exp(dt_t[None, :, :] * at[:, None, :])
        h = dA * h + (dt_t[None, :, :] * b_t[:, :, None]) * u_t[None, :, :]
        y = jnp.sum(h * c_t[:, :, None], axis=0)
        ys_ref[pl.ds(t, 1)] = y[None]
        return h

    def step10(i, h):
        t = i * 10
        for j in range(10):
            h = one_step(t + j, h)
        return h

    lax.fori_loop(0, L // 10, step10, jnp.zeros((D_STATE, _BB, D_INNER), f32))

    y = ys_ref[...] + xc * dd_ref[...].reshape(1, 1, D_INNER)
    y = y * (z * jax.nn.sigmoid(z))
    feats = jnp.dot(y.reshape(L * _BB, D_INNER), wout_ref[...],
                    preferred_element_type=f32)

    gt = lax.dot_general(gw_ref[...], feats, (((0,), (1,)), ((), ())),
                         preferred_element_type=f32)
    gt = gt + jnp.transpose(gb_ref[...])
    m = jnp.max(gt, axis=0, keepdims=True)
    ex = jnp.exp(gt - m)
    sm = ex / jnp.sum(ex, axis=0, keepdims=True)
    msm = jnp.max(sm, axis=0, keepdims=True)
    ism = sm == msm
    run = ism[0:1]
    wrows = [jnp.where(run, msm, 0.0)]
    for e in range(1, N_EXP):
        fe = ism[e:e + 1] & (~run)
        run = run | fe
        wrows.append(jnp.where(fe, msm, 0.0))
    wsel_t = jnp.concatenate(wrows, axis=0)
    wsel = jnp.transpose(wsel_t)
    col = lax.broadcasted_iota(jnp.int32, (N_EXP, N_EXP * H), 1)
    row = lax.broadcasted_iota(jnp.int32, (N_EXP, N_EXP * H), 0)
    ones_blk = jnp.where(col // H == row, 1.0, 0.0)
    bcast = jnp.dot(wsel, ones_blk, preferred_element_type=f32)
    eo = jnp.dot(feats, wexp_ref[...], preferred_element_type=f32) + bexp_ref[...]
    prod = bcast * eo
    out = (prod[:, 0:H] + prod[:, H:2 * H]) + (prod[:, 2 * H:3 * H]
                                               + prod[:, 3 * H:4 * H])
    moe = out.reshape(L, _BB, H)

    pos_o_ref[...] = jnp.sum(moe * pose_ref[...], axis=-1)[None]
    neg_o_ref[...] = jnp.sum(moe * nege_ref[...], axis=-1)[None]


def _dense_tc(seq_e, pos_e, neg_e, pemb, win, cwt, cb, wx4, wdt, bdt, wxbc,
              at, dd, wout, gw, gb, wexp, bexp):
    grid = (B_SZ // _BB,)
    big = pl.BlockSpec((L, _BB, H), lambda i: (0, i, 0))
    full = lambda shape: pl.BlockSpec(shape, lambda i: tuple(0 for _ in shape))
    outs = pl.BlockSpec((1, L, _BB), lambda i: (i, 0, 0))
    return pl.pallas_call(
        _tc_body,
        grid=grid,
        in_specs=[
            big, big, big,
            full((L, 1, H)),
            full((H, 2 * D_INNER)),
            full((D_CONV, D_INNER)),
            full((1, D_INNER)),
            full((D_INNER, 8)),
            full((8, D_INNER)),
            full((1, D_INNER)),
            full((D_INNER, 2 * D_STATE)),
            full((D_STATE, D_INNER)),
            full((1, D_INNER)),
            full((D_INNER, H)),
            full((H, N_EXP)),
            full((1, N_EXP)),
            full((H, N_EXP * H)),
            full((1, N_EXP * H)),
        ],
        out_specs=[outs, outs],
        out_shape=[jax.ShapeDtypeStruct((B_SZ // _BB, L, _BB), jnp.float32)] * 2,
        scratch_shapes=[pltpu.VMEM((L, _BB, D_INNER), jnp.float32),
                        pltpu.VMEM((L, _BB, D_INNER), jnp.float32),
                        pltpu.VMEM((L, _BB, D_INNER), jnp.float32),
                        pltpu.VMEM((L, 2 * D_STATE, _BB), jnp.float32)],
        compiler_params=pltpu.CompilerParams(
            dimension_semantics=("arbitrary",)),
    )(seq_e, pos_e, neg_e, pemb, win, cwt, cb, wx4, wdt, bdt, wxbc, at, dd,
      wout, gw, gb, wexp, bexp)


def kernel(user_ids, log_seqs, pos_seqs, neg_seqs, params):
    p = params
    idx = jnp.concatenate([
        log_seqs.T.reshape(-1), pos_seqs.T.reshape(-1), neg_seqs.T.reshape(-1)
    ]).astype(jnp.int32)
    rows = _gather_sc(p['item_emb'], idx)
    n = B_SZ * L
    seq_e = rows[:n].reshape(L, B_SZ, H)
    pos_e = rows[n:2 * n].reshape(L, B_SZ, H)
    neg_e = rows[2 * n:].reshape(L, B_SZ, H)

    dt_rank = p['W_dt'].shape[0]
    wx4 = jnp.pad(p['W_x'][:, :dt_rank], ((0, 0), (0, 8 - dt_rank)))
    wdt = jnp.pad(p['W_dt'], ((0, 8 - dt_rank), (0, 0)))
    wxbc = p['W_x'][:, dt_rank:]
    at = (-jnp.exp(p['A_log'])).T
    wexp = jnp.transpose(p['exp_W'], (1, 0, 2)).reshape(H, N_EXP * H)
    bexp = p['exp_b'].reshape(1, N_EXP * H)

    pos_l, neg_l = _dense_tc(
        seq_e, pos_e, neg_e,
        p['pos_emb'][:L].reshape(L, 1, H),
        p['W_in'],
        p['conv_w'].T,
        p['conv_b'].reshape(1, D_INNER),
        wx4,
        wdt,
        p['b_dt'].reshape(1, D_INNER),
        wxbc,
        at,
        p['D'].reshape(1, D_INNER),
        p['W_out'],
        p['gate_W'],
        p['gate_b'].reshape(1, N_EXP),
        wexp,
        bexp,
    )
    return (jnp.transpose(pos_l, (0, 2, 1)).reshape(B_SZ, L),
            jnp.transpose(neg_l, (0, 2, 1)).reshape(B_SZ, L))
